# baseline (speedup 1.0000x reference)
import jax
import jax.numpy as jnp
from jax.experimental import pallas as pl
from jax.experimental.pallas import tpu as pltpu

_CHUNK = 10000
_NBUF = 4


def _mm_bias_kernel(x_hbm, w_ref, b_ref, o_hbm, xbuf, obuf, *sems):
    in_sems = sems[:_NBUF]
    out_sems = sems[_NBUF:]
    i = pl.program_id(0)
    n = pl.num_programs(0)

    def in_copy(c, s):
        return pltpu.make_async_copy(
            x_hbm.at[pl.ds(c * _CHUNK, _CHUNK), :], xbuf.at[s], in_sems[s]
        )

    def out_copy(c, s):
        return pltpu.make_async_copy(
            obuf.at[s], o_hbm.at[pl.ds(c * _CHUNK, _CHUNK), :], out_sems[s]
        )

    @pl.when(i == 0)
    def _():
        for c in range(_NBUF - 1):
            in_copy(c, c).start()

    la = i + _NBUF - 1
    la_slot = la % _NBUF
    for s in range(_NBUF):
        @pl.when(jnp.logical_and(la_slot == s, la < n))
        def _(s=s):
            in_copy(la, s).start()

    slot = i % _NBUF
    for s in range(_NBUF):
        @pl.when(slot == s)
        def _(s=s):
            in_copy(i, s).wait()

            @pl.when(i >= _NBUF)
            def _():
                out_copy(i - _NBUF, s).wait()

            obuf[s] = (
                jnp.dot(xbuf[s], w_ref[...], preferred_element_type=jnp.float32)
                + b_ref[...]
            )
            out_copy(i, s).start()

    @pl.when(i == n - 1)
    def _():
        for d in range(min(_NBUF, n)):
            c = n - 1 - d
            out_copy(c, c % _NBUF).wait()


def kernel(input, kernel, bias):
    n, in_ch = input.shape
    out_ch = kernel.shape[1]
    nchunks = n // _CHUNK
    return pl.pallas_call(
        _mm_bias_kernel,
        grid=(nchunks,),
        in_specs=[
            pl.BlockSpec(memory_space=pl.ANY),
            pl.BlockSpec((in_ch, out_ch), lambda i: (0, 0)),
            pl.BlockSpec((1, out_ch), lambda i: (0, 0)),
        ],
        out_specs=pl.BlockSpec(memory_space=pl.ANY),
        out_shape=jax.ShapeDtypeStruct((n, out_ch), jnp.float32),
        scratch_shapes=(
            [
                pltpu.VMEM((_NBUF, _CHUNK, in_ch), jnp.float32),
                pltpu.VMEM((_NBUF, _CHUNK, out_ch), jnp.float32),
            ]
            + [pltpu.SemaphoreType.DMA] * (2 * _NBUF)
        ),
    )(input, kernel, bias)
